# baseline (speedup 1.0000x reference)
import functools

import jax
import jax.numpy as jnp
from jax import lax
from jax.experimental import pallas as pl
from jax.experimental.pallas import tpu as pltpu
from jax.experimental.pallas import tpu_sc as plsc

N = 10000
E = 320000
D = 128

NC = 2
NS = 16
NW = NC * NS
C = 128
NBLK = E // C
NB = NBLK // NW
EXTRA = NBLK - NB * NW
N_PAD = 10240
CNT_STRIPE = N_PAD // NS
ROW_STRIPE = 624
ROW_TAIL = N - NS * ROW_STRIPE

_mesh = plsc.VectorSubcoreMesh(core_axis_name="c", subcore_axis_name="s")


@functools.partial(
    pl.kernel,
    mesh=_mesh,
    out_type=[
        jax.ShapeDtypeStruct((NC, N, D), jnp.float32),
        jax.ShapeDtypeStruct((NC, N_PAD), jnp.float32),
    ],
    scratch_types=[
        pltpu.VMEM((C,), jnp.int32),
        pltpu.VMEM((C,), jnp.int32),
        pltpu.VMEM((C,), jnp.int32),
        pltpu.VMEM((C,), jnp.int32),
        pltpu.VMEM((C,), jnp.int32),
        pltpu.VMEM((C,), jnp.int32),
        pltpu.VMEM((C,), jnp.int32),
        pltpu.VMEM((C,), jnp.int32),
        pltpu.VMEM((C, D), jnp.float32),
        pltpu.VMEM((C, D), jnp.float32),
        pltpu.VMEM((C,), jnp.float32),
        pltpu.VMEM_SHARED((N, D), jnp.float32),
        pltpu.VMEM_SHARED((N_PAD,), jnp.float32),
        pltpu.SemaphoreType.DMA,
        pltpu.SemaphoreType.DMA,
        pltpu.SemaphoreType.DMA,
        pltpu.SemaphoreType.DMA,
        pltpu.SemaphoreType.DMA,
        pltpu.SemaphoreType.DMA,
        pltpu.SemaphoreType.DMA,
        pltpu.SemaphoreType.DMA,
    ],
)
def _aggregate(eix_hbm, x_hbm, zf_hbm, zc_hbm, sums_out, cnt_out,
               srcb0, srcb1, srcb2, srcb3, dstb0, dstb1, dstb2, dstb3,
               rows0, rows1, ones_v, acc_sh, cnt_sh,
               sg0, sg1, ss0, ss1, si0, si1, si2, si3):
    cid = lax.axis_index("c")
    sid = lax.axis_index("s")
    wid = sid * NC + cid

    r0 = sid * ROW_STRIPE
    pltpu.sync_copy(zf_hbm.at[pl.ds(r0, ROW_STRIPE)],
                    acc_sh.at[pl.ds(r0, ROW_STRIPE)])

    @pl.when(sid == NS - 1)
    def _zero_tail():
        pltpu.sync_copy(zf_hbm.at[pl.ds(NS * ROW_STRIPE, ROW_TAIL)],
                        acc_sh.at[pl.ds(NS * ROW_STRIPE, ROW_TAIL)])

    c0 = sid * CNT_STRIPE
    pltpu.sync_copy(zc_hbm.at[pl.ds(c0, CNT_STRIPE)],
                    cnt_sh.at[pl.ds(c0, CNT_STRIPE)])
    for i in range(C // 16):
        ones_v[pl.ds(i * 16, 16)] = jnp.ones((16,), jnp.float32)
    plsc.subcore_barrier()

    wblk = wid * NB
    srcb = [srcb0, srcb1, srcb2, srcb3]
    dstb = [dstb0, dstb1, dstb2, dstb3]
    rows = [rows0, rows1]
    sem_g = [sg0, sg1]
    sem_s = [ss0, ss1]
    sem_i = [si0, si1, si2, si3]

    def i_start(blk, q):
        off = pl.multiple_of(blk * C, 128)
        pltpu.async_copy(eix_hbm.at[0, pl.ds(off, C)], srcb[q], sem_i[q])
        pltpu.async_copy(eix_hbm.at[1, pl.ds(off, C)], dstb[q], sem_i[q])

    def i_wait(q):
        pltpu.make_async_copy(eix_hbm.at[0, pl.ds(0, C)], srcb[q], sem_i[q]).wait()
        pltpu.make_async_copy(eix_hbm.at[0, pl.ds(0, C)], dstb[q], sem_i[q]).wait()

    def g_start(p, q):
        pltpu.async_copy(x_hbm.at[srcb[q]], rows[p], sem_g[p])

    def g_wait(p):
        pltpu.make_async_copy(x_hbm.at[pl.ds(0, C)], rows[p], sem_g[p]).wait()

    def s_start(p, q):
        pltpu.async_copy(rows[p], acc_sh.at[dstb[q]], sem_s[p], add=True)
        pltpu.async_copy(ones_v, cnt_sh.at[dstb[q]], sem_s[p], add=True)

    def s_wait(p):
        pltpu.make_async_copy(x_hbm.at[pl.ds(0, C)], rows[p], sem_s[p]).wait()
        pltpu.make_async_copy(zc_hbm.at[pl.ds(0, C)], ones_v, sem_s[p]).wait()

    def chunk(j, p2, p4, first=False, issue_g=True, issue_i=True):
        if not first:
            s_wait(1 - p2)
        if issue_g:
            i_wait((p4 + 1) % 4)
            g_start(1 - p2, (p4 + 1) % 4)
        g_wait(p2)
        s_start(p2, p4)
        if issue_i:
            i_start(wblk + j + 3, (p4 + 3) % 4)

    i_start(wblk, 0)
    i_start(wblk + 1, 1)
    i_start(wblk + 2, 2)
    i_wait(0)
    g_start(0, 0)
    chunk(0, 0, 0, first=True)
    chunk(1, 1, 1)

    def body(t, carry):
        j = 4 * t + 2
        chunk(j, 0, 2)
        chunk(j + 1, 1, 3)
        chunk(j + 2, 0, 0)
        chunk(j + 3, 1, 1)
        return carry

    lax.fori_loop(0, (NB - 6) // 4, body, 0)
    chunk(NB - 4, 0, 2)
    chunk(NB - 3, 1, 3, issue_i=False)
    chunk(NB - 2, 0, 0, issue_i=False)
    chunk(NB - 1, 1, 1, issue_g=False, issue_i=False)
    s_wait(1)

    @pl.when(wid < EXTRA)
    def _extra_block():
        i_start(NB * NW + wid, 0)
        i_wait(0)
        g_start(0, 0)
        g_wait(0)
        s_start(0, 0)
        s_wait(0)

    plsc.subcore_barrier()

    pltpu.sync_copy(acc_sh.at[pl.ds(r0, ROW_STRIPE)],
                    sums_out.at[cid, pl.ds(r0, ROW_STRIPE)])

    @pl.when(sid == NS - 1)
    def _write_tail():
        pltpu.sync_copy(acc_sh.at[pl.ds(NS * ROW_STRIPE, ROW_TAIL)],
                        sums_out.at[cid, pl.ds(NS * ROW_STRIPE, ROW_TAIL)])

    pltpu.sync_copy(cnt_sh.at[pl.ds(c0, CNT_STRIPE)],
                    cnt_out.at[cid, pl.ds(c0, CNT_STRIPE)])


_BN = 2000


def _combine_body(s_ref, c_ref, x_ref, wl_ref, bl_ref, wr_ref, o_ref):
    sums = (s_ref[0] + s_ref[1]).reshape(_BN, D)
    cnt = c_ref[0] + c_ref[1]
    mean = sums / jnp.maximum(cnt, 1.0)
    dn = (((1,), (1,)), ((), ()))
    h = lax.dot_general(mean, wl_ref[...], dn,
                        preferred_element_type=jnp.float32)
    h = h + lax.dot_general(x_ref[...], wr_ref[...], dn,
                            preferred_element_type=jnp.float32)
    h = h + bl_ref[...]
    o_ref[...] = jnp.where(h > 0, h, 0.5 * h)


def _combine(sums, cnt, x, W_l, b_l, W_r):
    return pl.pallas_call(
        _combine_body,
        grid=(N // _BN,),
        in_specs=[
            pl.BlockSpec((NC, _BN * D), lambda i: (0, i)),
            pl.BlockSpec((NC, _BN, 1), lambda i: (0, i, 0)),
            pl.BlockSpec((_BN, D), lambda i: (i, 0)),
            pl.BlockSpec((D, D), lambda i: (0, 0)),
            pl.BlockSpec((1, D), lambda i: (0, 0)),
            pl.BlockSpec((D, D), lambda i: (0, 0)),
        ],
        out_specs=pl.BlockSpec((_BN, D), lambda i: (i, 0)),
        out_shape=jax.ShapeDtypeStruct((N, D), jnp.float32),
    )(sums.reshape(NC, N * D), cnt, x, W_l, b_l, W_r)


def kernel(x, edge_index, W_l, b_l, W_r):
    zf = jnp.zeros((N, D), jnp.float32)
    zc = jnp.zeros((N_PAD,), jnp.float32)
    sums, cnt = _aggregate(edge_index, x, zf, zc)
    h = _combine(sums, cnt[:, :N, None], x, W_l, b_l.reshape(1, D), W_r)
    return (h, x)


# speedup vs baseline: 1.0552x; 1.0552x over previous
import functools

import jax
import jax.numpy as jnp
from jax import lax
from jax.experimental import pallas as pl
from jax.experimental.pallas import tpu as pltpu
from jax.experimental.pallas import tpu_sc as plsc

N = 10000
E = 320000
D = 128

NC = 2
NS = 16
NW = NC * NS
C = 128
NBLK = E // C
NB = NBLK // NW
EXTRA = NBLK - NB * NW
N_PAD = 10240
CNT_STRIPE = N_PAD // NS
ROW_STRIPE = 624
ROW_TAIL = N - NS * ROW_STRIPE

_mesh = plsc.VectorSubcoreMesh(core_axis_name="c", subcore_axis_name="s")


@functools.partial(
    pl.kernel,
    mesh=_mesh,
    out_type=[
        jax.ShapeDtypeStruct((NC, N, D), jnp.float32),
        jax.ShapeDtypeStruct((NC, N_PAD), jnp.float32),
    ],
    scratch_types=[
        pltpu.VMEM((C,), jnp.int32),
        pltpu.VMEM((C,), jnp.int32),
        pltpu.VMEM((C,), jnp.int32),
        pltpu.VMEM((C,), jnp.int32),
        pltpu.VMEM((C,), jnp.int32),
        pltpu.VMEM((C,), jnp.int32),
        pltpu.VMEM((C,), jnp.int32),
        pltpu.VMEM((C,), jnp.int32),
        pltpu.VMEM((C, D), jnp.float32),
        pltpu.VMEM((C, D), jnp.float32),
        pltpu.VMEM((C,), jnp.float32),
        pltpu.VMEM_SHARED((N, D), jnp.float32),
        pltpu.VMEM_SHARED((N_PAD,), jnp.float32),
        pltpu.SemaphoreType.DMA,
        pltpu.SemaphoreType.DMA,
        pltpu.SemaphoreType.DMA,
        pltpu.SemaphoreType.DMA,
        pltpu.SemaphoreType.DMA,
        pltpu.SemaphoreType.DMA,
        pltpu.SemaphoreType.DMA,
        pltpu.SemaphoreType.DMA,
    ],
)
def _aggregate(eix_hbm, x_hbm, zf_hbm, zc_hbm, sums_out, cnt_out,
               srcb0, srcb1, srcb2, srcb3, dstb0, dstb1, dstb2, dstb3,
               rows0, rows1, ones_v, acc_sh, cnt_sh,
               sg0, sg1, ss0, ss1, si0, si1, si2, si3):
    cid = lax.axis_index("c")
    sid = lax.axis_index("s")
    wid = sid * NC + cid


    wblk = wid * NB
    srcb = [srcb0, srcb1, srcb2, srcb3]
    dstb = [dstb0, dstb1, dstb2, dstb3]
    rows = [rows0, rows1]
    sem_g = [sg0, sg1]
    sem_s = [ss0, ss1]
    sem_i = [si0, si1, si2, si3]

    def i_start(blk, q):
        off = pl.multiple_of(blk * C, 128)
        pltpu.async_copy(eix_hbm.at[0, pl.ds(off, C)], srcb[q], sem_i[q])
        pltpu.async_copy(eix_hbm.at[1, pl.ds(off, C)], dstb[q], sem_i[q])

    def i_wait(q):
        pltpu.make_async_copy(eix_hbm.at[0, pl.ds(0, C)], srcb[q], sem_i[q]).wait()
        pltpu.make_async_copy(eix_hbm.at[0, pl.ds(0, C)], dstb[q], sem_i[q]).wait()

    def g_start(p, q):
        pltpu.async_copy(x_hbm.at[srcb[q]], rows[p], sem_g[p])

    def g_wait(p):
        pltpu.make_async_copy(x_hbm.at[pl.ds(0, C)], rows[p], sem_g[p]).wait()

    def s_start(p, q):
        pltpu.async_copy(rows[p], acc_sh.at[dstb[q]], sem_s[p], add=True)
        pltpu.async_copy(ones_v, cnt_sh.at[dstb[q]], sem_s[p], add=True)

    def s_wait(p):
        pltpu.make_async_copy(x_hbm.at[pl.ds(0, C)], rows[p], sem_s[p]).wait()
        pltpu.make_async_copy(zc_hbm.at[pl.ds(0, C)], ones_v, sem_s[p]).wait()

    def chunk(j, p2, p4, first=False, issue_g=True, issue_i=True):
        if not first:
            s_wait(1 - p2)
        if issue_g:
            i_wait((p4 + 1) % 4)
            g_start(1 - p2, (p4 + 1) % 4)
        g_wait(p2)
        s_start(p2, p4)
        if issue_i:
            i_start(wblk + j + 3, (p4 + 3) % 4)

    i_start(wblk, 0)
    i_start(wblk + 1, 1)
    i_start(wblk + 2, 2)
    i_wait(0)
    g_start(0, 0)

    r0 = sid * ROW_STRIPE
    pltpu.sync_copy(zf_hbm.at[pl.ds(r0, ROW_STRIPE)],
                    acc_sh.at[pl.ds(r0, ROW_STRIPE)])

    @pl.when(sid == NS - 1)
    def _zero_tail():
        pltpu.sync_copy(zf_hbm.at[pl.ds(NS * ROW_STRIPE, ROW_TAIL)],
                        acc_sh.at[pl.ds(NS * ROW_STRIPE, ROW_TAIL)])

    c0 = sid * CNT_STRIPE
    pltpu.sync_copy(zc_hbm.at[pl.ds(c0, CNT_STRIPE)],
                    cnt_sh.at[pl.ds(c0, CNT_STRIPE)])
    for i in range(C // 16):
        ones_v[pl.ds(i * 16, 16)] = jnp.ones((16,), jnp.float32)
    plsc.subcore_barrier()

    chunk(0, 0, 0, first=True)
    chunk(1, 1, 1)

    def body(t, carry):
        j = 4 * t + 2
        chunk(j, 0, 2)
        chunk(j + 1, 1, 3)
        chunk(j + 2, 0, 0)
        chunk(j + 3, 1, 1)
        return carry

    lax.fori_loop(0, (NB - 6) // 4, body, 0)
    chunk(NB - 4, 0, 2)
    chunk(NB - 3, 1, 3, issue_i=False)
    chunk(NB - 2, 0, 0, issue_i=False)
    chunk(NB - 1, 1, 1, issue_g=False, issue_i=False)
    s_wait(1)

    @pl.when(wid < EXTRA)
    def _extra_block():
        i_start(NB * NW + wid, 0)
        i_wait(0)
        g_start(0, 0)
        g_wait(0)
        s_start(0, 0)
        s_wait(0)

    plsc.subcore_barrier()

    pltpu.sync_copy(acc_sh.at[pl.ds(r0, ROW_STRIPE)],
                    sums_out.at[cid, pl.ds(r0, ROW_STRIPE)])

    @pl.when(sid == NS - 1)
    def _write_tail():
        pltpu.sync_copy(acc_sh.at[pl.ds(NS * ROW_STRIPE, ROW_TAIL)],
                        sums_out.at[cid, pl.ds(NS * ROW_STRIPE, ROW_TAIL)])

    pltpu.sync_copy(cnt_sh.at[pl.ds(c0, CNT_STRIPE)],
                    cnt_out.at[cid, pl.ds(c0, CNT_STRIPE)])


_BN = 2000


def _combine_body(s_ref, c_ref, x_ref, wl_ref, bl_ref, wr_ref, o_ref):
    sums = s_ref[0] + s_ref[1]
    cnt = c_ref[0] + c_ref[1]
    mean = sums / jnp.maximum(cnt, 1.0)
    dn = (((1,), (1,)), ((), ()))
    h = lax.dot_general(mean, wl_ref[...], dn,
                        preferred_element_type=jnp.float32)
    h = h + lax.dot_general(x_ref[...], wr_ref[...], dn,
                            preferred_element_type=jnp.float32)
    h = h + bl_ref[...]
    o_ref[...] = jnp.where(h > 0, h, 0.5 * h)


def _combine(sums, cnt, x, W_l, b_l, W_r):
    return pl.pallas_call(
        _combine_body,
        grid=(N // _BN,),
        in_specs=[
            pl.BlockSpec((NC, _BN, D), lambda i: (0, i, 0)),
            pl.BlockSpec((NC, _BN, 1), lambda i: (0, i, 0)),
            pl.BlockSpec((_BN, D), lambda i: (i, 0)),
            pl.BlockSpec((D, D), lambda i: (0, 0)),
            pl.BlockSpec((1, D), lambda i: (0, 0)),
            pl.BlockSpec((D, D), lambda i: (0, 0)),
        ],
        out_specs=pl.BlockSpec((_BN, D), lambda i: (i, 0)),
        out_shape=jax.ShapeDtypeStruct((N, D), jnp.float32),
    )(sums, cnt, x, W_l, b_l, W_r)


def kernel(x, edge_index, W_l, b_l, W_r):
    zf = jnp.zeros((N, D), jnp.float32)
    zc = jnp.zeros((N_PAD,), jnp.float32)
    sums, cnt = _aggregate(edge_index, x, zf, zc)
    h = _combine(sums, cnt[:, :N, None], x, W_l, b_l.reshape(1, D), W_r)
    return (h, x)
